# baseline (speedup 1.0000x reference)
import jax
import jax.numpy as jnp
from jax import lax
from jax.experimental import pallas as pl
from jax.experimental.pallas import tpu as pltpu
from jax.experimental.pallas import tpu_sc as plsc

N = 6400000
NNODES = 100000

NC = 2
NS = 16
NW = NC * NS
L = 16
U1 = 5
U3 = 10

NPAD = 102400
R1 = N // NS
C1 = 4000
R3 = N // NW
C3 = 4000
E2 = NPAD // NS


def _mesh():
    return plsc.VectorSubcoreMesh(core_axis_name="c", subcore_axis_name="s")


def _strict_group(table, idxv, vv):
    cc = plsc.load_gather(table, [idxv])
    nv = jnp.maximum(cc, vv)
    plsc.store_scatter(table, [idxv], nv)
    c2 = plsc.load_gather(table, [idxv])
    pend = (c2 < nv).astype(jnp.int32)

    @pl.when(jnp.max(pend, axis=0) > 0)
    def _retry():
        def body(_, p):
            m = p > 0
            c3 = plsc.load_gather(table, [idxv])
            n3 = jnp.maximum(c3, vv)
            plsc.store_scatter(table, [idxv], n3, mask=m)
            c4 = plsc.load_gather(table, [idxv])
            return ((c4 < n3) & m).astype(jnp.int32)

        lax.fori_loop(0, L - 1, body, pend)


def _segmax_chunk(table, idxbuf, valbuf):
    def pack(p, carry):
        gb = p * (U1 * L)
        idxs = [idxbuf[pl.ds(gb + u * L, L)] for u in range(U1)]
        vals = [valbuf[pl.ds(gb + u * L, L)] for u in range(U1)]
        curs = [plsc.load_gather(table, [idxs[u]]) for u in range(U1)]
        news = [jnp.maximum(curs[u], vals[u]) for u in range(U1)]
        for u in range(U1):
            plsc.store_scatter(table, [idxs[u]], news[u])
        chk = [plsc.load_gather(table, [idxs[u]]) for u in range(U1)]
        bad = (chk[0] < news[0])
        for u in range(1, U1):
            bad = bad | (chk[u] < news[u])

        @pl.when(jnp.any(bad))
        def _repair():
            for u in range(U1):
                _strict_group(table, idxs[u], vals[u])

        return carry

    lax.fori_loop(0, C1 // (U1 * L), pack, 0)


def _k1_body(ids, vals, zeros, partials, table, idxA, valA, idxB, valB,
             semA, semB):
    c = lax.axis_index("c")
    s = lax.axis_index("s")
    pltpu.sync_copy(zeros, table)
    base = c * N + s * R1
    vbase = s * R1
    nch = R1 // C1

    def start(bufi, bufv, sem, k):
        kc = jnp.minimum(k, nch - 1) * C1
        pltpu.async_copy(ids.at[pl.ds(base + kc, C1)], bufi, sem)
        pltpu.async_copy(vals.at[pl.ds(vbase + kc, C1)], bufv, sem)

    def wait(bufi, bufv, sem, k):
        kc = jnp.minimum(k, nch - 1) * C1
        pltpu.make_async_copy(ids.at[pl.ds(base + kc, C1)], bufi, sem).wait()
        pltpu.make_async_copy(vals.at[pl.ds(vbase + kc, C1)], bufv, sem).wait()

    start(idxA, valA, semA, 0)
    start(idxB, valB, semB, 1)

    def body(k2, carry):
        k0 = 2 * k2
        wait(idxA, valA, semA, k0)
        _segmax_chunk(table, idxA, valA)
        start(idxA, valA, semA, k0 + 2)
        wait(idxB, valB, semB, k0 + 1)
        _segmax_chunk(table, idxB, valB)
        start(idxB, valB, semB, k0 + 3)
        return carry

    lax.fori_loop(0, nch // 2, body, 0)
    wait(idxA, valA, semA, nch - 1)
    wait(idxB, valB, semB, nch - 1)
    pltpu.sync_copy(table, partials.at[pl.ds((c * NS + s) * NPAD, NPAD)])


def _k2_body(partials, tables, acc, tmp):
    c = lax.axis_index("c")
    s = lax.axis_index("s")
    seg = s * E2
    pltpu.sync_copy(partials.at[pl.ds(c * NS * NPAD + seg, E2)], acc)

    def one_partial(p, carry):
        pltpu.sync_copy(partials.at[pl.ds((c * NS + p) * NPAD + seg, E2)], tmp)

        def grp(g, carry2):
            for u in range(5):
                d = pl.ds((g * 5 + u) * L, L)
                acc[d] = jnp.maximum(acc[d], tmp[d])
            return carry2

        lax.fori_loop(0, E2 // (5 * L), grp, 0)
        return carry

    lax.fori_loop(1, NS, one_partial, 0)
    pltpu.sync_copy(acc, tables.at[pl.ds(c * NPAD + seg, E2)])


def _gather_chunk(tbl, idxbuf, gbuf):
    def pack(p, carry):
        gb = p * (U3 * L)
        for u in range(U3):
            d = pl.ds(gb + u * L, L)
            gbuf[d] = plsc.load_gather(tbl, [idxbuf[d]])
        return carry

    lax.fori_loop(0, C3 // (U3 * L), pack, 0)


def _gather_mul_chunk(tbl, idxbuf, g0buf, gbuf):
    def pack(p, carry):
        gb = p * (U3 * L)
        for u in range(U3):
            d = pl.ds(gb + u * L, L)
            gbuf[d] = plsc.load_gather(tbl, [idxbuf[d]]) * g0buf[d]
        return carry

    lax.fori_loop(0, C3 // (U3 * L), pack, 0)


def _k3_body(ids, tables, out, gtmp, dump, tbl, idxA, idxB, gA, gB, g0A, g0B,
             semA, semB, semOA, semOB):
    c = lax.axis_index("c")
    s = lax.axis_index("s")
    w = s * NC + c
    base = w * R3
    nch = R3 // C3

    def startp0(bufi, sem, k):
        kc = base + jnp.minimum(k, nch - 1) * C3
        pltpu.async_copy(ids.at[pl.ds(kc, C3)], bufi, sem)

    def waitp0(bufi, sem, k):
        kc = base + jnp.minimum(k, nch - 1) * C3
        pltpu.make_async_copy(ids.at[pl.ds(kc, C3)], bufi, sem).wait()

    def prime_out():
        pltpu.async_copy(gA, dump.at[pl.ds(w * 2 * C3, C3)], semOA)
        pltpu.async_copy(gB, dump.at[pl.ds(w * 2 * C3 + C3, C3)], semOB)

    pltpu.sync_copy(tables.at[pl.ds(0, NPAD)], tbl)
    startp0(idxA, semA, 0)
    startp0(idxB, semB, 1)
    prime_out()

    def body0(k2, carry):
        k0 = 2 * k2
        off0 = base + k0 * C3
        waitp0(idxA, semA, k0)
        pltpu.make_async_copy(gA, gtmp.at[pl.ds(off0, C3)], semOA).wait()
        _gather_chunk(tbl, idxA, gA)
        pltpu.async_copy(gA, gtmp.at[pl.ds(off0, C3)], semOA)
        startp0(idxA, semA, k0 + 2)
        waitp0(idxB, semB, k0 + 1)
        pltpu.make_async_copy(gB, gtmp.at[pl.ds(off0 + C3, C3)], semOB).wait()
        _gather_chunk(tbl, idxB, gB)
        pltpu.async_copy(gB, gtmp.at[pl.ds(off0 + C3, C3)], semOB)
        startp0(idxB, semB, k0 + 3)
        return carry

    lax.fori_loop(0, nch // 2, body0, 0)
    waitp0(idxA, semA, nch - 1)
    waitp0(idxB, semB, nch - 1)
    pltpu.make_async_copy(gA, gtmp.at[pl.ds(base, C3)], semOA).wait()
    pltpu.make_async_copy(gB, gtmp.at[pl.ds(base, C3)], semOB).wait()

    pltpu.sync_copy(tables.at[pl.ds(NPAD, NPAD)], tbl)

    def startp1(bufi, bufg, semi, k):
        kc = jnp.minimum(k, nch - 1) * C3
        pltpu.async_copy(ids.at[pl.ds(N + base + kc, C3)], bufi, semi)
        pltpu.async_copy(gtmp.at[pl.ds(base + kc, C3)], bufg, semi)

    def waitp1(bufi, bufg, semi, k):
        kc = jnp.minimum(k, nch - 1) * C3
        pltpu.make_async_copy(ids.at[pl.ds(N + base + kc, C3)], bufi, semi).wait()
        pltpu.make_async_copy(gtmp.at[pl.ds(base + kc, C3)], bufg, semi).wait()

    startp1(idxA, g0A, semA, 0)
    startp1(idxB, g0B, semB, 1)
    prime_out()

    def body1(k2, carry):
        k0 = 2 * k2
        off0 = base + k0 * C3
        waitp1(idxA, g0A, semA, k0)
        pltpu.make_async_copy(gA, out.at[pl.ds(off0, C3)], semOA).wait()
        _gather_mul_chunk(tbl, idxA, g0A, gA)
        pltpu.async_copy(gA, out.at[pl.ds(off0, C3)], semOA)
        startp1(idxA, g0A, semA, k0 + 2)
        waitp1(idxB, g0B, semB, k0 + 1)
        pltpu.make_async_copy(gB, out.at[pl.ds(off0 + C3, C3)], semOB).wait()
        _gather_mul_chunk(tbl, idxB, g0B, gB)
        pltpu.async_copy(gB, out.at[pl.ds(off0 + C3, C3)], semOB)
        startp1(idxB, g0B, semB, k0 + 3)
        return carry

    lax.fori_loop(0, nch // 2, body1, 0)
    waitp1(idxA, g0A, semA, nch - 1)
    waitp1(idxB, g0B, semB, nch - 1)
    pltpu.make_async_copy(gA, out.at[pl.ds(base, C3)], semOA).wait()
    pltpu.make_async_copy(gB, out.at[pl.ds(base, C3)], semOB).wait()


def _k1():
    return pl.kernel(
        _k1_body,
        out_type=jax.ShapeDtypeStruct((NC * NS * NPAD,), jnp.float32),
        mesh=_mesh(),
        compiler_params=pltpu.CompilerParams(needs_layout_passes=False),
        scratch_types=[
            pltpu.VMEM((NPAD,), jnp.float32),
            pltpu.VMEM((C1,), jnp.int32),
            pltpu.VMEM((C1,), jnp.float32),
            pltpu.VMEM((C1,), jnp.int32),
            pltpu.VMEM((C1,), jnp.float32),
            pltpu.SemaphoreType.DMA,
            pltpu.SemaphoreType.DMA,
        ],
    )


def _k2():
    return pl.kernel(
        _k2_body,
        out_type=jax.ShapeDtypeStruct((NC * NPAD,), jnp.float32),
        mesh=_mesh(),
        compiler_params=pltpu.CompilerParams(needs_layout_passes=False),
        scratch_types=[
            pltpu.VMEM((E2,), jnp.float32),
            pltpu.VMEM((E2,), jnp.float32),
        ],
    )


def _k3():
    return pl.kernel(
        _k3_body,
        out_type=(
            jax.ShapeDtypeStruct((N,), jnp.float32),
            jax.ShapeDtypeStruct((N,), jnp.float32),
            jax.ShapeDtypeStruct((NW * 2 * C3,), jnp.float32),
        ),
        mesh=_mesh(),
        compiler_params=pltpu.CompilerParams(needs_layout_passes=False),
        scratch_types=[
            pltpu.VMEM((NPAD,), jnp.float32),
            pltpu.VMEM((C3,), jnp.int32),
            pltpu.VMEM((C3,), jnp.int32),
            pltpu.VMEM((C3,), jnp.float32),
            pltpu.VMEM((C3,), jnp.float32),
            pltpu.VMEM((C3,), jnp.float32),
            pltpu.VMEM((C3,), jnp.float32),
            pltpu.SemaphoreType.DMA,
            pltpu.SemaphoreType.DMA,
            pltpu.SemaphoreType.DMA,
            pltpu.SemaphoreType.DMA,
        ],
    )


@jax.jit
def kernel(pred_pair, reg_feat):
    ids = pred_pair.T.reshape(NC * N)
    zeros = jnp.zeros((NPAD,), jnp.float32)
    partials = _k1()(ids, reg_feat, zeros)
    tables = _k2()(partials)
    out, _, _ = _k3()(ids, tables)
    return out


# speedup vs baseline: 1.2183x; 1.2183x over previous
import jax
import jax.numpy as jnp
from jax import lax
from jax.experimental import pallas as pl
from jax.experimental.pallas import tpu as pltpu
from jax.experimental.pallas import tpu_sc as plsc

N = 6400000
NNODES = 100000

NC = 2
NS = 16
NW = NC * NS
L = 16
U1 = 5
U3 = 5

NPAD = 102400
R1 = N // NS
C1 = 4000
R3 = N // NW
C3 = 4000
E2 = NPAD // NS


def _mesh():
    return plsc.VectorSubcoreMesh(core_axis_name="c", subcore_axis_name="s")


def _strict_group(table, idxv, vv):
    cc = plsc.load_gather(table, [idxv])
    nv = jnp.maximum(cc, vv)
    plsc.store_scatter(table, [idxv], nv)
    c2 = plsc.load_gather(table, [idxv])
    pend = (c2 < nv).astype(jnp.int32)

    @pl.when(jnp.max(pend, axis=0) > 0)
    def _retry():
        def body(_, p):
            m = p > 0
            c3 = plsc.load_gather(table, [idxv])
            n3 = jnp.maximum(c3, vv)
            plsc.store_scatter(table, [idxv], n3, mask=m)
            c4 = plsc.load_gather(table, [idxv])
            return ((c4 < n3) & m).astype(jnp.int32)

        lax.fori_loop(0, L - 1, body, pend)


def _segmax_chunk(table, idxbuf, valbuf):
    def pack(p, carry):
        gb = p * (U1 * L)
        idxs = [idxbuf[pl.ds(gb + u * L, L)] for u in range(U1)]
        vals = [valbuf[pl.ds(gb + u * L, L)] for u in range(U1)]
        curs = [plsc.load_gather(table, [idxs[u]]) for u in range(U1)]
        news = [jnp.maximum(curs[u], vals[u]) for u in range(U1)]
        for u in range(U1):
            plsc.store_scatter(table, [idxs[u]], news[u])
        chk = [plsc.load_gather(table, [idxs[u]]) for u in range(U1)]
        bad = (chk[0] < news[0])
        for u in range(1, U1):
            bad = bad | (chk[u] < news[u])

        @pl.when(jnp.any(bad))
        def _repair():
            for u in range(U1):
                _strict_group(table, idxs[u], vals[u])

        return carry

    lax.fori_loop(0, C1 // (U1 * L), pack, 0)


def _k1_body(ids, vals, zeros, partials, table, idxA, valA, idxB, valB,
             semA, semB):
    c = lax.axis_index("c")
    s = lax.axis_index("s")
    pltpu.sync_copy(zeros, table)
    base = c * N + s * R1
    vbase = s * R1
    nch = R1 // C1

    def start(bufi, bufv, sem, k):
        kc = jnp.minimum(k, nch - 1) * C1
        pltpu.async_copy(ids.at[pl.ds(base + kc, C1)], bufi, sem)
        pltpu.async_copy(vals.at[pl.ds(vbase + kc, C1)], bufv, sem)

    def wait(bufi, bufv, sem, k):
        kc = jnp.minimum(k, nch - 1) * C1
        pltpu.make_async_copy(ids.at[pl.ds(base + kc, C1)], bufi, sem).wait()
        pltpu.make_async_copy(vals.at[pl.ds(vbase + kc, C1)], bufv, sem).wait()

    start(idxA, valA, semA, 0)
    start(idxB, valB, semB, 1)

    def body(k2, carry):
        k0 = 2 * k2
        wait(idxA, valA, semA, k0)
        _segmax_chunk(table, idxA, valA)
        start(idxA, valA, semA, k0 + 2)
        wait(idxB, valB, semB, k0 + 1)
        _segmax_chunk(table, idxB, valB)
        start(idxB, valB, semB, k0 + 3)
        return carry

    lax.fori_loop(0, nch // 2, body, 0)
    wait(idxA, valA, semA, nch - 1)
    wait(idxB, valB, semB, nch - 1)
    pltpu.sync_copy(table, partials.at[pl.ds((c * NS + s) * NPAD, NPAD)])


def _k2_body(partials, tables, acc, tmp):
    c = lax.axis_index("c")
    s = lax.axis_index("s")
    seg = s * E2
    pltpu.sync_copy(partials.at[pl.ds(c * NS * NPAD + seg, E2)], acc)

    def one_partial(p, carry):
        pltpu.sync_copy(partials.at[pl.ds((c * NS + p) * NPAD + seg, E2)], tmp)

        def grp(g, carry2):
            for u in range(5):
                d = pl.ds((g * 5 + u) * L, L)
                acc[d] = jnp.maximum(acc[d], tmp[d])
            return carry2

        lax.fori_loop(0, E2 // (5 * L), grp, 0)
        return carry

    lax.fori_loop(1, NS, one_partial, 0)
    pltpu.sync_copy(acc, tables.at[pl.ds(c * NPAD + seg, E2)])


def _gather_chunk(tbl, idxbuf, gbuf):
    def pack(p, carry):
        gb = p * (U3 * L)
        for u in range(U3):
            d = pl.ds(gb + u * L, L)
            gbuf[d] = plsc.load_gather(tbl, [idxbuf[d]])
        return carry

    lax.fori_loop(0, C3 // (U3 * L), pack, 0)


def _gather_mul_chunk(tbl, idxbuf, g0buf, gbuf):
    def pack(p, carry):
        gb = p * (U3 * L)
        for u in range(U3):
            d = pl.ds(gb + u * L, L)
            gbuf[d] = plsc.load_gather(tbl, [idxbuf[d]]) * g0buf[d]
        return carry

    lax.fori_loop(0, C3 // (U3 * L), pack, 0)


def _k3_body(ids, tables, out, gtmp, dump, tbl, idxA, idxB, gA, gB, g0A, g0B,
             semA, semB, semOA, semOB):
    c = lax.axis_index("c")
    s = lax.axis_index("s")
    w = s * NC + c
    base = w * R3
    nch = R3 // C3

    def startp0(bufi, sem, k):
        kc = base + jnp.minimum(k, nch - 1) * C3
        pltpu.async_copy(ids.at[pl.ds(kc, C3)], bufi, sem)

    def waitp0(bufi, sem, k):
        kc = base + jnp.minimum(k, nch - 1) * C3
        pltpu.make_async_copy(ids.at[pl.ds(kc, C3)], bufi, sem).wait()

    def prime_out():
        pltpu.async_copy(gA, dump.at[pl.ds(w * 2 * C3, C3)], semOA)
        pltpu.async_copy(gB, dump.at[pl.ds(w * 2 * C3 + C3, C3)], semOB)

    pltpu.sync_copy(tables.at[pl.ds(0, NPAD)], tbl)
    startp0(idxA, semA, 0)
    startp0(idxB, semB, 1)
    prime_out()

    def body0(k2, carry):
        k0 = 2 * k2
        off0 = base + k0 * C3
        waitp0(idxA, semA, k0)
        pltpu.make_async_copy(gA, gtmp.at[pl.ds(off0, C3)], semOA).wait()
        _gather_chunk(tbl, idxA, gA)
        pltpu.async_copy(gA, gtmp.at[pl.ds(off0, C3)], semOA)
        startp0(idxA, semA, k0 + 2)
        waitp0(idxB, semB, k0 + 1)
        pltpu.make_async_copy(gB, gtmp.at[pl.ds(off0 + C3, C3)], semOB).wait()
        _gather_chunk(tbl, idxB, gB)
        pltpu.async_copy(gB, gtmp.at[pl.ds(off0 + C3, C3)], semOB)
        startp0(idxB, semB, k0 + 3)
        return carry

    lax.fori_loop(0, nch // 2, body0, 0)
    waitp0(idxA, semA, nch - 1)
    waitp0(idxB, semB, nch - 1)
    pltpu.make_async_copy(gA, gtmp.at[pl.ds(base, C3)], semOA).wait()
    pltpu.make_async_copy(gB, gtmp.at[pl.ds(base, C3)], semOB).wait()

    pltpu.sync_copy(tables.at[pl.ds(NPAD, NPAD)], tbl)

    def startp1(bufi, bufg, semi, k):
        kc = jnp.minimum(k, nch - 1) * C3
        pltpu.async_copy(ids.at[pl.ds(N + base + kc, C3)], bufi, semi)
        pltpu.async_copy(gtmp.at[pl.ds(base + kc, C3)], bufg, semi)

    def waitp1(bufi, bufg, semi, k):
        kc = jnp.minimum(k, nch - 1) * C3
        pltpu.make_async_copy(ids.at[pl.ds(N + base + kc, C3)], bufi, semi).wait()
        pltpu.make_async_copy(gtmp.at[pl.ds(base + kc, C3)], bufg, semi).wait()

    startp1(idxA, g0A, semA, 0)
    startp1(idxB, g0B, semB, 1)
    prime_out()

    def body1(k2, carry):
        k0 = 2 * k2
        off0 = base + k0 * C3
        waitp1(idxA, g0A, semA, k0)
        pltpu.make_async_copy(gA, out.at[pl.ds(off0, C3)], semOA).wait()
        _gather_mul_chunk(tbl, idxA, g0A, gA)
        pltpu.async_copy(gA, out.at[pl.ds(off0, C3)], semOA)
        startp1(idxA, g0A, semA, k0 + 2)
        waitp1(idxB, g0B, semB, k0 + 1)
        pltpu.make_async_copy(gB, out.at[pl.ds(off0 + C3, C3)], semOB).wait()
        _gather_mul_chunk(tbl, idxB, g0B, gB)
        pltpu.async_copy(gB, out.at[pl.ds(off0 + C3, C3)], semOB)
        startp1(idxB, g0B, semB, k0 + 3)
        return carry

    lax.fori_loop(0, nch // 2, body1, 0)
    waitp1(idxA, g0A, semA, nch - 1)
    waitp1(idxB, g0B, semB, nch - 1)
    pltpu.make_async_copy(gA, out.at[pl.ds(base, C3)], semOA).wait()
    pltpu.make_async_copy(gB, out.at[pl.ds(base, C3)], semOB).wait()


def _k1():
    return pl.kernel(
        _k1_body,
        out_type=jax.ShapeDtypeStruct((NC * NS * NPAD,), jnp.float32),
        mesh=_mesh(),
        compiler_params=pltpu.CompilerParams(needs_layout_passes=False),
        scratch_types=[
            pltpu.VMEM((NPAD,), jnp.float32),
            pltpu.VMEM((C1,), jnp.int32),
            pltpu.VMEM((C1,), jnp.float32),
            pltpu.VMEM((C1,), jnp.int32),
            pltpu.VMEM((C1,), jnp.float32),
            pltpu.SemaphoreType.DMA,
            pltpu.SemaphoreType.DMA,
        ],
    )


def _k2():
    return pl.kernel(
        _k2_body,
        out_type=jax.ShapeDtypeStruct((NC * NPAD,), jnp.float32),
        mesh=_mesh(),
        compiler_params=pltpu.CompilerParams(needs_layout_passes=False),
        scratch_types=[
            pltpu.VMEM((E2,), jnp.float32),
            pltpu.VMEM((E2,), jnp.float32),
        ],
    )


def _k3():
    return pl.kernel(
        _k3_body,
        out_type=(
            jax.ShapeDtypeStruct((N,), jnp.float32),
            jax.ShapeDtypeStruct((N,), jnp.float32),
            jax.ShapeDtypeStruct((NW * 2 * C3,), jnp.float32),
        ),
        mesh=_mesh(),
        compiler_params=pltpu.CompilerParams(needs_layout_passes=False),
        scratch_types=[
            pltpu.VMEM((NPAD,), jnp.float32),
            pltpu.VMEM((C3,), jnp.int32),
            pltpu.VMEM((C3,), jnp.int32),
            pltpu.VMEM((C3,), jnp.float32),
            pltpu.VMEM((C3,), jnp.float32),
            pltpu.VMEM((C3,), jnp.float32),
            pltpu.VMEM((C3,), jnp.float32),
            pltpu.SemaphoreType.DMA,
            pltpu.SemaphoreType.DMA,
            pltpu.SemaphoreType.DMA,
            pltpu.SemaphoreType.DMA,
        ],
    )


@jax.jit
def kernel(pred_pair, reg_feat):
    ids = pred_pair.T.reshape(NC * N)
    zeros = jnp.zeros((NPAD,), jnp.float32)
    partials = _k1()(ids, reg_feat, zeros)
    tables = _k2()(partials)
    out, _, _ = _k3()(ids, tables)
    return out


# speedup vs baseline: 1.2307x; 1.0102x over previous
import jax
import jax.numpy as jnp
from jax import lax
from jax.experimental import pallas as pl
from jax.experimental.pallas import tpu as pltpu
from jax.experimental.pallas import tpu_sc as plsc

N = 6400000
NNODES = 100000

NC = 2
NS = 16
NW = NC * NS
L = 16
U1 = 5
U3 = 5

NPAD = 102400
R1 = N // NS
C1 = 4000
R3 = N // NW
C3 = 4000
E2 = NPAD // NS


def _mesh():
    return plsc.VectorSubcoreMesh(core_axis_name="c", subcore_axis_name="s")


def _strict_group(table, idxv, vv):
    cc = plsc.load_gather(table, [idxv])
    nv = jnp.maximum(cc, vv)
    plsc.store_scatter(table, [idxv], nv)
    c2 = plsc.load_gather(table, [idxv])
    pend = (c2 < nv).astype(jnp.int32)

    @pl.when(jnp.max(pend, axis=0) > 0)
    def _retry():
        def body(_, p):
            m = p > 0
            c3 = plsc.load_gather(table, [idxv])
            n3 = jnp.maximum(c3, vv)
            plsc.store_scatter(table, [idxv], n3, mask=m)
            c4 = plsc.load_gather(table, [idxv])
            return ((c4 < n3) & m).astype(jnp.int32)

        lax.fori_loop(0, L - 1, body, pend)


def _segmax_chunk(table, idxbuf, valbuf):
    def pack(p, carry):
        gb = p * (U1 * L)
        idxs = [idxbuf[pl.ds(gb + u * L, L)] for u in range(U1)]
        vals = [valbuf[pl.ds(gb + u * L, L)] for u in range(U1)]
        curs = [plsc.load_gather(table, [idxs[u]]) for u in range(U1)]
        news = [jnp.maximum(curs[u], vals[u]) for u in range(U1)]
        for u in range(U1):
            plsc.store_scatter(table, [idxs[u]], news[u])
        chk = [plsc.load_gather(table, [idxs[u]]) for u in range(U1)]
        bad = (chk[0] < news[0])
        for u in range(1, U1):
            bad = bad | (chk[u] < news[u])

        @pl.when(jnp.any(bad))
        def _repair():
            for u in range(U1):
                _strict_group(table, idxs[u], vals[u])

        return carry

    lax.fori_loop(0, C1 // (U1 * L), pack, 0)


def _k1_body(ids, vals, zeros, partials, table, idxA, valA, idxB, valB,
             semA, semB):
    c = lax.axis_index("c")
    s = lax.axis_index("s")
    pltpu.sync_copy(zeros, table)
    base = c * N + s * R1
    vbase = s * R1
    nch = R1 // C1

    def start(bufi, bufv, sem, k):
        kc = jnp.minimum(k, nch - 1) * C1
        pltpu.async_copy(ids.at[pl.ds(base + kc, C1)], bufi, sem)
        pltpu.async_copy(vals.at[pl.ds(vbase + kc, C1)], bufv, sem)

    def wait(bufi, bufv, sem, k):
        kc = jnp.minimum(k, nch - 1) * C1
        pltpu.make_async_copy(ids.at[pl.ds(base + kc, C1)], bufi, sem).wait()
        pltpu.make_async_copy(vals.at[pl.ds(vbase + kc, C1)], bufv, sem).wait()

    start(idxA, valA, semA, 0)
    start(idxB, valB, semB, 1)

    def body(k2, carry):
        k0 = 2 * k2
        wait(idxA, valA, semA, k0)
        _segmax_chunk(table, idxA, valA)
        start(idxA, valA, semA, k0 + 2)
        wait(idxB, valB, semB, k0 + 1)
        _segmax_chunk(table, idxB, valB)
        start(idxB, valB, semB, k0 + 3)
        return carry

    lax.fori_loop(0, nch // 2, body, 0)
    wait(idxA, valA, semA, nch - 1)
    wait(idxB, valB, semB, nch - 1)
    pltpu.sync_copy(table, partials.at[pl.ds((c * NS + s) * NPAD, NPAD)])


def _k2_body(partials, tables, acc, tmpA, tmpB, semA, semB):
    c = lax.axis_index("c")
    s = lax.axis_index("s")
    seg = s * E2
    bufs = [(tmpA, semA), (tmpB, semB)]

    def src(p):
        return partials.at[pl.ds((c * NS + p) * NPAD + seg, E2)]

    pltpu.sync_copy(src(0), acc)
    pltpu.async_copy(src(1), tmpA, semA)
    for p in range(1, NS):
        tmp, sem = bufs[(p - 1) % 2]
        pltpu.make_async_copy(src(p), tmp, sem).wait()
        if p + 1 < NS:
            nxt, nsem = bufs[p % 2]
            pltpu.async_copy(src(p + 1), nxt, nsem)

        def grp(g, carry2, tmp=tmp):
            for u in range(5):
                d = pl.ds((g * 5 + u) * L, L)
                acc[d] = jnp.maximum(acc[d], tmp[d])
            return carry2

        lax.fori_loop(0, E2 // (5 * L), grp, 0)
    pltpu.sync_copy(acc, tables.at[pl.ds(c * NPAD + seg, E2)])


def _gather_chunk(tbl, idxbuf, gbuf):
    def pack(p, carry):
        gb = p * (U3 * L)
        for u in range(U3):
            d = pl.ds(gb + u * L, L)
            gbuf[d] = plsc.load_gather(tbl, [idxbuf[d]])
        return carry

    lax.fori_loop(0, C3 // (U3 * L), pack, 0)


def _gather_mul_chunk(tbl, idxbuf, g0buf, gbuf):
    def pack(p, carry):
        gb = p * (U3 * L)
        for u in range(U3):
            d = pl.ds(gb + u * L, L)
            gbuf[d] = plsc.load_gather(tbl, [idxbuf[d]]) * g0buf[d]
        return carry

    lax.fori_loop(0, C3 // (U3 * L), pack, 0)


def _k3_body(ids, tables, out, gtmp, dump, tbl, idxA, idxB, gA, gB, g0A, g0B,
             semA, semB, semOA, semOB):
    c = lax.axis_index("c")
    s = lax.axis_index("s")
    w = s * NC + c
    base = w * R3
    nch = R3 // C3

    def startp0(bufi, sem, k):
        kc = base + jnp.minimum(k, nch - 1) * C3
        pltpu.async_copy(ids.at[pl.ds(kc, C3)], bufi, sem)

    def waitp0(bufi, sem, k):
        kc = base + jnp.minimum(k, nch - 1) * C3
        pltpu.make_async_copy(ids.at[pl.ds(kc, C3)], bufi, sem).wait()

    def prime_out():
        pltpu.async_copy(gA, dump.at[pl.ds(w * 2 * C3, C3)], semOA)
        pltpu.async_copy(gB, dump.at[pl.ds(w * 2 * C3 + C3, C3)], semOB)

    pltpu.sync_copy(tables.at[pl.ds(0, NPAD)], tbl)
    startp0(idxA, semA, 0)
    startp0(idxB, semB, 1)
    prime_out()

    def body0(k2, carry):
        k0 = 2 * k2
        off0 = base + k0 * C3
        waitp0(idxA, semA, k0)
        pltpu.make_async_copy(gA, gtmp.at[pl.ds(off0, C3)], semOA).wait()
        _gather_chunk(tbl, idxA, gA)
        pltpu.async_copy(gA, gtmp.at[pl.ds(off0, C3)], semOA)
        startp0(idxA, semA, k0 + 2)
        waitp0(idxB, semB, k0 + 1)
        pltpu.make_async_copy(gB, gtmp.at[pl.ds(off0 + C3, C3)], semOB).wait()
        _gather_chunk(tbl, idxB, gB)
        pltpu.async_copy(gB, gtmp.at[pl.ds(off0 + C3, C3)], semOB)
        startp0(idxB, semB, k0 + 3)
        return carry

    lax.fori_loop(0, nch // 2, body0, 0)
    waitp0(idxA, semA, nch - 1)
    waitp0(idxB, semB, nch - 1)
    pltpu.make_async_copy(gA, gtmp.at[pl.ds(base, C3)], semOA).wait()
    pltpu.make_async_copy(gB, gtmp.at[pl.ds(base, C3)], semOB).wait()

    pltpu.sync_copy(tables.at[pl.ds(NPAD, NPAD)], tbl)

    def startp1(bufi, bufg, semi, k):
        kc = jnp.minimum(k, nch - 1) * C3
        pltpu.async_copy(ids.at[pl.ds(N + base + kc, C3)], bufi, semi)
        pltpu.async_copy(gtmp.at[pl.ds(base + kc, C3)], bufg, semi)

    def waitp1(bufi, bufg, semi, k):
        kc = jnp.minimum(k, nch - 1) * C3
        pltpu.make_async_copy(ids.at[pl.ds(N + base + kc, C3)], bufi, semi).wait()
        pltpu.make_async_copy(gtmp.at[pl.ds(base + kc, C3)], bufg, semi).wait()

    startp1(idxA, g0A, semA, 0)
    startp1(idxB, g0B, semB, 1)
    prime_out()

    def body1(k2, carry):
        k0 = 2 * k2
        off0 = base + k0 * C3
        waitp1(idxA, g0A, semA, k0)
        pltpu.make_async_copy(gA, out.at[pl.ds(off0, C3)], semOA).wait()
        _gather_mul_chunk(tbl, idxA, g0A, gA)
        pltpu.async_copy(gA, out.at[pl.ds(off0, C3)], semOA)
        startp1(idxA, g0A, semA, k0 + 2)
        waitp1(idxB, g0B, semB, k0 + 1)
        pltpu.make_async_copy(gB, out.at[pl.ds(off0 + C3, C3)], semOB).wait()
        _gather_mul_chunk(tbl, idxB, g0B, gB)
        pltpu.async_copy(gB, out.at[pl.ds(off0 + C3, C3)], semOB)
        startp1(idxB, g0B, semB, k0 + 3)
        return carry

    lax.fori_loop(0, nch // 2, body1, 0)
    waitp1(idxA, g0A, semA, nch - 1)
    waitp1(idxB, g0B, semB, nch - 1)
    pltpu.make_async_copy(gA, out.at[pl.ds(base, C3)], semOA).wait()
    pltpu.make_async_copy(gB, out.at[pl.ds(base, C3)], semOB).wait()


def _k1():
    return pl.kernel(
        _k1_body,
        out_type=jax.ShapeDtypeStruct((NC * NS * NPAD,), jnp.float32),
        mesh=_mesh(),
        compiler_params=pltpu.CompilerParams(needs_layout_passes=False),
        scratch_types=[
            pltpu.VMEM((NPAD,), jnp.float32),
            pltpu.VMEM((C1,), jnp.int32),
            pltpu.VMEM((C1,), jnp.float32),
            pltpu.VMEM((C1,), jnp.int32),
            pltpu.VMEM((C1,), jnp.float32),
            pltpu.SemaphoreType.DMA,
            pltpu.SemaphoreType.DMA,
        ],
    )


def _k2():
    return pl.kernel(
        _k2_body,
        out_type=jax.ShapeDtypeStruct((NC * NPAD,), jnp.float32),
        mesh=_mesh(),
        compiler_params=pltpu.CompilerParams(needs_layout_passes=False),
        scratch_types=[
            pltpu.VMEM((E2,), jnp.float32),
            pltpu.VMEM((E2,), jnp.float32),
            pltpu.VMEM((E2,), jnp.float32),
            pltpu.SemaphoreType.DMA,
            pltpu.SemaphoreType.DMA,
        ],
    )


def _k3():
    return pl.kernel(
        _k3_body,
        out_type=(
            jax.ShapeDtypeStruct((N,), jnp.float32),
            jax.ShapeDtypeStruct((N,), jnp.float32),
            jax.ShapeDtypeStruct((NW * 2 * C3,), jnp.float32),
        ),
        mesh=_mesh(),
        compiler_params=pltpu.CompilerParams(needs_layout_passes=False),
        scratch_types=[
            pltpu.VMEM((NPAD,), jnp.float32),
            pltpu.VMEM((C3,), jnp.int32),
            pltpu.VMEM((C3,), jnp.int32),
            pltpu.VMEM((C3,), jnp.float32),
            pltpu.VMEM((C3,), jnp.float32),
            pltpu.VMEM((C3,), jnp.float32),
            pltpu.VMEM((C3,), jnp.float32),
            pltpu.SemaphoreType.DMA,
            pltpu.SemaphoreType.DMA,
            pltpu.SemaphoreType.DMA,
            pltpu.SemaphoreType.DMA,
        ],
    )


@jax.jit
def kernel(pred_pair, reg_feat):
    ids = pred_pair.T.reshape(NC * N)
    zeros = jnp.zeros((NPAD,), jnp.float32)
    partials = _k1()(ids, reg_feat, zeros)
    tables = _k2()(partials)
    out, _, _ = _k3()(ids, tables)
    return out
